# baseline (speedup 1.0000x reference)
import functools

import jax
import jax.numpy as jnp
from jax import lax
from jax.experimental import pallas as pl
from jax.experimental.pallas import tpu as pltpu
from jax.experimental.pallas import tpu_sc as plsc

N_BATCH = 16
N_CHAN = 512
N_COL = 4096

R = N_BATCH * N_CHAN
NC = 2
NS = 16
NW = NC * NS
ROWS_PER_TILE = R // NW
K = 8
CHUNKS = ROWS_PER_TILE // K

_mesh = plsc.VectorSubcoreMesh(core_axis_name="c", subcore_axis_name="s")


@functools.partial(
    pl.kernel,
    mesh=_mesh,
    out_type=jax.ShapeDtypeStruct((R, N_COL), jnp.float32),
    scratch_types=[
        pltpu.SemaphoreType.DMA,
    ],
)
def _reverse_rows(x_hbm, idx_hbm, out_hbm, sem):
    del idx_hbm
    wid = lax.axis_index("s") * NC + lax.axis_index("c")
    base = wid * ROWS_PER_TILE

    def body(r, carry):
        dst = base + r
        src = lax.bitwise_xor(dst, N_CHAN - 1)
        pltpu.async_copy(x_hbm.at[src], out_hbm.at[dst], sem)
        return carry

    lax.fori_loop(0, ROWS_PER_TILE, body, 0)
    pltpu.make_async_copy(
        x_hbm.at[pl.ds(0, ROWS_PER_TILE)],
        out_hbm.at[pl.ds(base, ROWS_PER_TILE)],
        sem,
    ).wait()


def kernel(x, cond):
    del cond
    xf = x.reshape(R, N_COL)
    idx = jnp.bitwise_xor(jnp.arange(R, dtype=jnp.int32), N_CHAN - 1)
    z = _reverse_rows(xf, idx)
    log_det_J = jnp.zeros((1,), dtype=jnp.float32)
    return (z.reshape(N_BATCH, N_CHAN, N_COL), log_det_J)


# speedup vs baseline: 35.0023x; 35.0023x over previous
import functools

import jax
import jax.numpy as jnp
from jax import lax
from jax.experimental import pallas as pl
from jax.experimental.pallas import tpu as pltpu
from jax.experimental.pallas import tpu_sc as plsc

N_BATCH = 16
N_CHAN = 512
N_COL = 4096

R = N_BATCH * N_CHAN
NC = 2
NS = 16
NW = NC * NS
ROWS_PER_TILE = R // NW
K = 8
CHUNKS = ROWS_PER_TILE // K

_mesh = plsc.VectorSubcoreMesh(core_axis_name="c", subcore_axis_name="s")


NBUF = 3


@functools.partial(
    pl.kernel,
    mesh=_mesh,
    out_type=jax.ShapeDtypeStruct((R, N_COL), jnp.float32),
    scratch_types=[
        pltpu.VMEM((ROWS_PER_TILE,), jnp.int32),
        pltpu.VMEM((NBUF, K, N_COL), jnp.float32),
        pltpu.SemaphoreType.DMA((NBUF,)),
        pltpu.SemaphoreType.DMA((NBUF,)),
    ],
)
def _reverse_rows(x_hbm, idx_hbm, out_hbm, idx_v, bufs, gsem, ssem):
    wid = lax.axis_index("s") * NC + lax.axis_index("c")
    base = wid * ROWS_PER_TILE
    pltpu.sync_copy(idx_hbm.at[pl.ds(base, ROWS_PER_TILE)], idx_v)

    gathers = [None] * CHUNKS
    stores = [None] * CHUNKS

    def fire_gather(c):
        b = c % NBUF
        if stores[c - NBUF] is not None:
            stores[c - NBUF].wait()
        gathers[c] = pltpu.async_copy(
            x_hbm.at[idx_v.at[pl.ds(c * K, K)]], bufs.at[b], gsem.at[b])

    fire_gather(0)
    fire_gather(1)
    for c in range(CHUNKS):
        if c + 2 < CHUNKS:
            fire_gather(c + 2)
        b = c % NBUF
        gathers[c].wait()
        stores[c] = pltpu.async_copy(
            bufs.at[b], out_hbm.at[pl.ds(base + c * K, K)], ssem.at[b])
    for c in range(CHUNKS - NBUF, CHUNKS):
        stores[c].wait()


def kernel(x, cond):
    del cond
    xf = x.reshape(R, N_COL)
    idx = jnp.bitwise_xor(jnp.arange(R, dtype=jnp.int32), N_CHAN - 1)
    z = _reverse_rows(xf, idx)
    log_det_J = jnp.zeros((1,), dtype=jnp.float32)
    return (z.reshape(N_BATCH, N_CHAN, N_COL), log_det_J)


# speedup vs baseline: 35.0347x; 1.0009x over previous
import functools

import numpy as np
import jax
import jax.numpy as jnp
from jax import lax
from jax.experimental import pallas as pl
from jax.experimental.pallas import tpu as pltpu
from jax.experimental.pallas import tpu_sc as plsc

N_BATCH = 16
N_CHAN = 512
N_COL = 4096

R = N_BATCH * N_CHAN
NC = 2
NS = 16
NW = NC * NS
ROWS_PER_TILE = R // NW
K = 8
CHUNKS = ROWS_PER_TILE // K

_mesh = plsc.VectorSubcoreMesh(core_axis_name="c", subcore_axis_name="s")

_IDX_NP = np.bitwise_xor(np.arange(R, dtype=np.int32), N_CHAN - 1)


NBUF = 3


@functools.partial(
    pl.kernel,
    mesh=_mesh,
    out_type=jax.ShapeDtypeStruct((R, N_COL), jnp.float32),
    scratch_types=[
        pltpu.VMEM((ROWS_PER_TILE,), jnp.int32),
        pltpu.VMEM((NBUF, K, N_COL), jnp.float32),
        pltpu.SemaphoreType.DMA((NBUF,)),
        pltpu.SemaphoreType.DMA((NBUF,)),
    ],
)
def _reverse_rows(x_hbm, idx_hbm, out_hbm, idx_v, bufs, gsem, ssem):
    wid = lax.axis_index("s") * NC + lax.axis_index("c")
    base = wid * ROWS_PER_TILE
    pltpu.sync_copy(idx_hbm.at[pl.ds(base, ROWS_PER_TILE)], idx_v)

    gathers = [None] * CHUNKS
    stores = [None] * CHUNKS

    def fire_gather(c):
        b = c % NBUF
        if stores[c - NBUF] is not None:
            stores[c - NBUF].wait()
        gathers[c] = pltpu.async_copy(
            x_hbm.at[idx_v.at[pl.ds(c * K, K)]], bufs.at[b], gsem.at[b])

    fire_gather(0)
    fire_gather(1)
    for c in range(CHUNKS):
        if c + 2 < CHUNKS:
            fire_gather(c + 2)
        b = c % NBUF
        gathers[c].wait()
        stores[c] = pltpu.async_copy(
            bufs.at[b], out_hbm.at[pl.ds(base + c * K, K)], ssem.at[b])
    for c in range(CHUNKS - NBUF, CHUNKS):
        stores[c].wait()


def kernel(x, cond):
    del cond
    xf = x.reshape(R, N_COL)
    idx = jnp.asarray(_IDX_NP)
    z = _reverse_rows(xf, idx)
    log_det_J = jnp.zeros((1,), dtype=jnp.float32)
    return (z.reshape(N_BATCH, N_CHAN, N_COL), log_det_J)
